# Initial kernel scaffold; baseline (speedup 1.0000x reference)
import functools

import jax
import jax.numpy as jnp
from jax import lax
from jax.experimental import pallas as pl
from jax.experimental.pallas import tpu as pltpu
from jax.experimental.pallas import tpu_sc as plsc

N_NODES = 10000
N_EDGES = 320000
NC = 2
NS = 16
NW = NC * NS
CH = 128
N_CHUNKS = N_EDGES // CH
ROWS_PER_SUB = N_NODES // NS

ROW_BLK = 1250


def _mm1_body(f_ref, w_ref, n_ref, o_ref):
    t = jnp.dot(f_ref[...], w_ref[...], preferred_element_type=jnp.float32)
    o_ref[...] = t * n_ref[...]


def _mm1(features, W, norm):
    return pl.pallas_call(
        _mm1_body,
        grid=(N_NODES // ROW_BLK,),
        in_specs=[
            pl.BlockSpec((ROW_BLK, features.shape[1]), lambda i: (i, 0)),
            pl.BlockSpec(W.shape, lambda i: (0, 0)),
            pl.BlockSpec((ROW_BLK, 1), lambda i: (i, 0)),
        ],
        out_specs=pl.BlockSpec((ROW_BLK, W.shape[1]), lambda i: (i, 0)),
        out_shape=jax.ShapeDtypeStruct((N_NODES, W.shape[1]), jnp.float32),
    )(features, W, norm)


def _mm2_body(p_ref, b_ref, w_ref, n_ref, o_ref):
    h = p_ref[0] + p_ref[1] + b_ref[...]
    h = jnp.maximum(h, 0.0)
    t = jnp.dot(h, w_ref[...], preferred_element_type=jnp.float32)
    o_ref[...] = t * n_ref[...]


def _mm2(partials, b, W, norm):
    d_in = partials.shape[2]
    return pl.pallas_call(
        _mm2_body,
        grid=(N_NODES // ROW_BLK,),
        in_specs=[
            pl.BlockSpec((2, ROW_BLK, d_in), lambda i: (0, i, 0)),
            pl.BlockSpec((1, d_in), lambda i: (0, 0)),
            pl.BlockSpec(W.shape, lambda i: (0, 0)),
            pl.BlockSpec((ROW_BLK, 1), lambda i: (i, 0)),
        ],
        out_specs=pl.BlockSpec((ROW_BLK, W.shape[1]), lambda i: (i, 0)),
        out_shape=jax.ShapeDtypeStruct((N_NODES, W.shape[1]), jnp.float32),
    )(partials, b.reshape(1, d_in), W, norm)


def _final_body(p_ref, b_ref, o_ref):
    o_ref[...] = p_ref[0] + p_ref[1] + b_ref[...]


def _final(partials, b):
    d = partials.shape[2]
    return pl.pallas_call(
        _final_body,
        grid=(N_NODES // ROW_BLK,),
        in_specs=[
            pl.BlockSpec((2, ROW_BLK, d), lambda i: (0, i, 0)),
            pl.BlockSpec((1, d), lambda i: (0, 0)),
        ],
        out_specs=pl.BlockSpec((ROW_BLK, d), lambda i: (i, 0)),
        out_shape=jax.ShapeDtypeStruct((N_NODES, d), jnp.float32),
    )(partials, b.reshape(1, d))


@functools.lru_cache(maxsize=None)
def _make_agg(feat):
    mesh = plsc.VectorSubcoreMesh(core_axis_name="c", subcore_axis_name="s")

    @functools.partial(
        pl.kernel,
        out_type=jax.ShapeDtypeStruct((NC, N_NODES, feat), jnp.float32),
        mesh=mesh,
        scratch_types=[
            pltpu.VMEM((CH,), jnp.int32),
            pltpu.VMEM((CH,), jnp.int32),
            pltpu.VMEM((CH, feat), jnp.float32),
            pltpu.VMEM_SHARED((N_NODES, feat), jnp.float32),
            pltpu.SemaphoreType.DMA,
        ],
    )
    def agg(t_hbm, src_hbm, dst_hbm, zero_hbm, out_hbm,
            idx_s, idx_d, rows, accum, sem):
        c = lax.axis_index("c")
        s = lax.axis_index("s")
        wid = s * NC + c
        sl = pl.ds(s * ROWS_PER_SUB, ROWS_PER_SUB)
        pltpu.sync_copy(zero_hbm, accum.at[sl])
        plsc.subcore_barrier()

        @pl.loop(wid, N_CHUNKS, step=NW)
        def _(r):
            base = r * CH
            pltpu.sync_copy(src_hbm.at[pl.ds(base, CH)], idx_s)
            pltpu.sync_copy(dst_hbm.at[pl.ds(base, CH)], idx_d)
            pltpu.async_copy(t_hbm.at[idx_s], rows, sem).wait()
            pltpu.sync_copy(rows, accum.at[idx_d], add=True)

        plsc.subcore_barrier()
        pltpu.sync_copy(accum.at[sl], out_hbm.at[c, sl])

    return agg


def _agg(t, src, dst):
    feat = t.shape[1]
    zero = jnp.zeros((ROWS_PER_SUB, feat), jnp.float32)
    return _make_agg(feat)(t, src, dst, zero)


def kernel(features, edge_index, norm, W1, b1, W2, b2, W3, b3):
    src = edge_index[0]
    dst = edge_index[1]
    t1 = _mm1(features, W1, norm)
    p1 = _agg(t1, src, dst)
    t2 = _mm2(p1, b1, W2, norm)
    p2 = _agg(t2, src, dst)
    t3 = _mm2(p2, b2, W3, norm)
    p3 = _agg(t3, src, dst)
    return _final(p3, b3)


# speedup vs baseline: 6.0260x; 6.0260x over previous
import functools

import jax
import jax.numpy as jnp
from jax import lax
from jax.experimental import pallas as pl
from jax.experimental.pallas import tpu as pltpu
from jax.experimental.pallas import tpu_sc as plsc

N_NODES = 10000
N_EDGES = 320000
NC = 2
NS = 16
NW = NC * NS
CH = 128
N_CHUNKS = N_EDGES // CH
ROWS_MAIN = 624
ROWS_LAST = N_NODES - 15 * ROWS_MAIN

ROW_BLK = 2000


def _mm1_body(f_ref, w_ref, n_ref, o_ref):
    t = jnp.dot(f_ref[...], w_ref[...], preferred_element_type=jnp.float32)
    o_ref[...] = t * n_ref[...]


def _mm1(features, W, norm):
    return pl.pallas_call(
        _mm1_body,
        grid=(N_NODES // ROW_BLK,),
        in_specs=[
            pl.BlockSpec((ROW_BLK, features.shape[1]), lambda i: (i, 0)),
            pl.BlockSpec(W.shape, lambda i: (0, 0)),
            pl.BlockSpec((ROW_BLK, 1), lambda i: (i, 0)),
        ],
        out_specs=pl.BlockSpec((ROW_BLK, W.shape[1]), lambda i: (i, 0)),
        out_shape=jax.ShapeDtypeStruct((N_NODES, W.shape[1]), jnp.float32),
    )(features, W, norm)


def _mm2_body(p_ref, b_ref, w_ref, n_ref, o_ref):
    h = p_ref[0] + p_ref[1] + b_ref[...]
    h = jnp.maximum(h, 0.0)
    t = jnp.dot(h, w_ref[...], preferred_element_type=jnp.float32)
    o_ref[...] = t * n_ref[...]


def _mm2(partials, b, W, norm):
    d_in = partials.shape[2]
    return pl.pallas_call(
        _mm2_body,
        grid=(N_NODES // ROW_BLK,),
        in_specs=[
            pl.BlockSpec((2, ROW_BLK, d_in), lambda i: (0, i, 0)),
            pl.BlockSpec((1, d_in), lambda i: (0, 0)),
            pl.BlockSpec(W.shape, lambda i: (0, 0)),
            pl.BlockSpec((ROW_BLK, 1), lambda i: (i, 0)),
        ],
        out_specs=pl.BlockSpec((ROW_BLK, W.shape[1]), lambda i: (i, 0)),
        out_shape=jax.ShapeDtypeStruct((N_NODES, W.shape[1]), jnp.float32),
    )(partials, b.reshape(1, d_in), W, norm)


def _scale_body(p_ref, b_ref, n_ref, o_ref):
    h = p_ref[0] + p_ref[1] + b_ref[...]
    o_ref[...] = jnp.maximum(h, 0.0) * n_ref[...]


def _scale(partials, b, norm):
    d = partials.shape[2]
    return pl.pallas_call(
        _scale_body,
        grid=(N_NODES // ROW_BLK,),
        in_specs=[
            pl.BlockSpec((2, ROW_BLK, d), lambda i: (0, i, 0)),
            pl.BlockSpec((1, d), lambda i: (0, 0)),
            pl.BlockSpec((ROW_BLK, 1), lambda i: (i, 0)),
        ],
        out_specs=pl.BlockSpec((ROW_BLK, d), lambda i: (i, 0)),
        out_shape=jax.ShapeDtypeStruct((N_NODES, d), jnp.float32),
    )(partials, b.reshape(1, d), norm)


def _final_body(p_ref, w_ref, b_ref, o_ref):
    h = p_ref[0] + p_ref[1]
    o_ref[...] = (
        jnp.dot(h, w_ref[...], preferred_element_type=jnp.float32) + b_ref[...]
    )


def _final(partials, W, b):
    d_in = partials.shape[2]
    d_out = W.shape[1]
    return pl.pallas_call(
        _final_body,
        grid=(N_NODES // ROW_BLK,),
        in_specs=[
            pl.BlockSpec((2, ROW_BLK, d_in), lambda i: (0, i, 0)),
            pl.BlockSpec(W.shape, lambda i: (0, 0)),
            pl.BlockSpec((1, d_out), lambda i: (0, 0)),
        ],
        out_specs=pl.BlockSpec((ROW_BLK, d_out), lambda i: (i, 0)),
        out_shape=jax.ShapeDtypeStruct((N_NODES, d_out), jnp.float32),
    )(partials, W, b.reshape(1, d_out))


@functools.lru_cache(maxsize=None)
def _make_agg(feat):
    mesh = plsc.VectorSubcoreMesh(core_axis_name="c", subcore_axis_name="s")

    @functools.partial(
        pl.kernel,
        out_type=jax.ShapeDtypeStruct((NC, N_NODES, feat), jnp.float32),
        mesh=mesh,
        scratch_types=[
            pltpu.VMEM((CH,), jnp.int32),
            pltpu.VMEM((CH,), jnp.int32),
            pltpu.VMEM((CH, feat), jnp.float32),
            pltpu.VMEM_SHARED((N_NODES, feat), jnp.float32),
            pltpu.SemaphoreType.DMA,
        ],
    )
    def agg(t_hbm, src_hbm, dst_hbm, zero_hbm, out_hbm,
            idx_s, idx_d, rows, accum, sem):
        c = lax.axis_index("c")
        s = lax.axis_index("s")
        wid = s * NC + c
        row0 = pl.multiple_of(s * ROWS_MAIN, 8)
        @pl.when(s < NS - 1)
        def _():
            pltpu.sync_copy(zero_hbm.at[pl.ds(0, ROWS_MAIN)],
                            accum.at[pl.ds(row0, ROWS_MAIN)])

        @pl.when(s == NS - 1)
        def _():
            pltpu.sync_copy(zero_hbm, accum.at[pl.ds(row0, ROWS_LAST)])

        plsc.subcore_barrier()

        @pl.loop(wid, N_CHUNKS, step=NW)
        def _(r):
            base = pl.multiple_of(r * CH, 8)
            pltpu.sync_copy(src_hbm.at[pl.ds(base, CH)], idx_s)
            pltpu.sync_copy(dst_hbm.at[pl.ds(base, CH)], idx_d)
            pltpu.async_copy(t_hbm.at[idx_s], rows, sem).wait()
            pltpu.sync_copy(rows, accum.at[idx_d], add=True)

        plsc.subcore_barrier()

        @pl.when(s < NS - 1)
        def _():
            pltpu.sync_copy(accum.at[pl.ds(row0, ROWS_MAIN)],
                            out_hbm.at[c, pl.ds(row0, ROWS_MAIN)])

        @pl.when(s == NS - 1)
        def _():
            pltpu.sync_copy(accum.at[pl.ds(row0, ROWS_LAST)],
                            out_hbm.at[c, pl.ds(row0, ROWS_LAST)])

    return agg


def _agg(t, src, dst):
    feat = t.shape[1]
    zero = jnp.zeros((ROWS_LAST, feat), jnp.float32)
    return _make_agg(feat)(t, src, dst, zero)


def kernel(features, edge_index, norm, W1, b1, W2, b2, W3, b3):
    src = edge_index[0]
    dst = edge_index[1]
    t1 = _mm1(features, W1, norm)
    p1 = _agg(t1, src, dst)
    t2 = _mm2(p1, b1, W2, norm)
    p2 = _agg(t2, src, dst)
    t3 = _scale(p2, b2, norm)
    p3 = _agg(t3, src, dst)
    return _final(p3, W3, b3)
